# Initial kernel scaffold; baseline (speedup 1.0000x reference)
import jax
import jax.numpy as jnp
from jax.experimental import pallas as pl


def kernel(x, edge_index, emb, W1, b1, W2, b2, W3, b3):
    raise NotImplementedError("write your pallas kernel here")


# speedup vs baseline: 14.2520x; 14.2520x over previous
import functools

import jax
import jax.numpy as jnp
from jax import lax
from jax.experimental import pallas as pl
from jax.experimental.pallas import tpu as pltpu
from jax.experimental.pallas import tpu_sc as plsc

N = 10000
E = 320000
NC, NS = 2, 16
NW = NC * NS
EW = E // NW
CH = 100
NCHUNK = EW // CH
FS = 64
RPT = N // NS

_mesh = plsc.VectorSubcoreMesh(
    core_axis_name="c", subcore_axis_name="s", num_cores=NC, num_subcores=NS)


@functools.partial(
    pl.kernel,
    out_type=jax.ShapeDtypeStruct((NW, RPT, 16), jnp.float32),
    mesh=_mesh,
    compiler_params=pltpu.CompilerParams(use_tc_tiling_on_sc=False),
    scratch_types=[
        pltpu.VMEM((NCHUNK, CH), jnp.int32),
        pltpu.VMEM((CH, 16), jnp.float32),
        pltpu.VMEM_SHARED((N, 16), jnp.float32),
    ],
)
def _deg_kernel(col_hbm, ones_hbm, zeros_hbm, out_hbm, cidx, ones_v, acc):
    c = lax.axis_index("c")
    s = lax.axis_index("s")
    w = c * NS + s
    pltpu.sync_copy(zeros_hbm, acc.at[pl.ds(s * RPT, RPT)])
    pltpu.sync_copy(col_hbm.at[w], cidx)
    pltpu.sync_copy(ones_hbm, ones_v)
    plsc.subcore_barrier()

    def body(j, carry):
        pltpu.sync_copy(ones_v, acc.at[cidx.at[j]], add=True)
        return carry

    lax.fori_loop(0, NCHUNK, body, 0)
    plsc.subcore_barrier()
    pltpu.sync_copy(acc.at[pl.ds(s * RPT, RPT)], out_hbm.at[w])


def _make_propagate(F):
    @functools.partial(
        pl.kernel,
        out_type=jax.ShapeDtypeStruct((NW, RPT, F), jnp.float32),
        mesh=_mesh,
        compiler_params=pltpu.CompilerParams(use_tc_tiling_on_sc=False),
        scratch_types=[
            pltpu.VMEM((NCHUNK, CH), jnp.int32),
            pltpu.VMEM((NCHUNK, CH), jnp.int32),
            pltpu.VMEM((CH, F), jnp.float32),
            pltpu.VMEM((CH, F), jnp.float32),
            pltpu.VMEM_SHARED((N, F), jnp.float32),
            pltpu.SemaphoreType.DMA,
            pltpu.SemaphoreType.DMA,
        ],
    )
    def _prop(g_hbm, row_hbm, col_hbm, zeros_hbm, out_hbm,
              ridx, cidx, buf0, buf1, acc, sem0, sem1):
        c = lax.axis_index("c")
        s = lax.axis_index("s")
        w = c * NS + s
        pltpu.sync_copy(zeros_hbm, acc.at[pl.ds(s * RPT, RPT)])
        pltpu.sync_copy(row_hbm.at[w], ridx)
        pltpu.sync_copy(col_hbm.at[w], cidx)
        pltpu.async_copy(g_hbm.at[ridx.at[0]], buf0, sem0)
        pltpu.async_copy(g_hbm.at[ridx.at[1]], buf1, sem1)
        plsc.subcore_barrier()

        def body(i, carry):
            j = 2 * i
            pltpu.make_async_copy(g_hbm.at[ridx.at[j]], buf0, sem0).wait()
            pltpu.sync_copy(buf0, acc.at[cidx.at[j]], add=True)
            pltpu.async_copy(g_hbm.at[ridx.at[j + 2]], buf0, sem0)
            pltpu.make_async_copy(g_hbm.at[ridx.at[j + 1]], buf1, sem1).wait()
            pltpu.sync_copy(buf1, acc.at[cidx.at[j + 1]], add=True)
            pltpu.async_copy(g_hbm.at[ridx.at[j + 3]], buf1, sem1)
            return carry

        lax.fori_loop(0, NCHUNK // 2 - 1, body, 0)
        j = NCHUNK - 2
        pltpu.make_async_copy(g_hbm.at[ridx.at[j]], buf0, sem0).wait()
        pltpu.sync_copy(buf0, acc.at[cidx.at[j]], add=True)
        pltpu.make_async_copy(g_hbm.at[ridx.at[j + 1]], buf1, sem1).wait()
        pltpu.sync_copy(buf1, acc.at[cidx.at[j + 1]], add=True)
        plsc.subcore_barrier()
        pltpu.sync_copy(acc.at[pl.ds(s * RPT, RPT)], out_hbm.at[w])

    return _prop


_prop = _make_propagate(FS)


def _dis_body(p_ref, o_ref):
    o_ref[...] = lax.rsqrt(1.0 + p_ref[0:1] + p_ref[1:2])


def _dis_tc(partials):
    return pl.pallas_call(
        _dis_body,
        out_shape=jax.ShapeDtypeStruct((1, N), jnp.float32),
    )(partials)


def _mm_body(dis_ref, x_ref, w_ref, o_ref):
    o_ref[...] = jnp.dot(dis_ref[...] * x_ref[...], w_ref[...],
                         preferred_element_type=jnp.float32)


def _mm_tc(dis, x, W, rb=2000):
    fin, fout = W.shape
    return pl.pallas_call(
        _mm_body,
        grid=(N // rb,),
        in_specs=[
            pl.BlockSpec((rb, 1), lambda i: (i, 0)),
            pl.BlockSpec((rb, fin), lambda i: (i, 0)),
            pl.BlockSpec((fin, fout), lambda i: (0, 0)),
        ],
        out_specs=pl.BlockSpec((rb, fout), lambda i: (i, 0)),
        out_shape=jax.ShapeDtypeStruct((N, fout), jnp.float32),
    )(dis, x, W)


def _comb_body(relu, s_ref, g_ref, dis_ref, b_ref, o_ref):
    v = dis_ref[...] * (s_ref[0] + s_ref[1] + g_ref[...]) + b_ref[...]
    o_ref[...] = jnp.maximum(v, 0.0) if relu else v


def _comb_tc(S, g, dis, b, relu, rb=2000):
    fout = g.shape[1]
    return pl.pallas_call(
        functools.partial(_comb_body, relu),
        grid=(N // rb,),
        in_specs=[
            pl.BlockSpec((NC, rb, fout), lambda i: (0, i, 0)),
            pl.BlockSpec((rb, fout), lambda i: (i, 0)),
            pl.BlockSpec((rb, 1), lambda i: (i, 0)),
            pl.BlockSpec((1, fout), lambda i: (0, 0)),
        ],
        out_specs=pl.BlockSpec((rb, fout), lambda i: (i, 0)),
        out_shape=jax.ShapeDtypeStruct((N, fout), jnp.float32),
    )(S, g, dis, b)


def kernel(x, edge_index, emb, W1, b1, W2, b2, W3, b3):
    ei = edge_index.astype(jnp.int32)
    row = ei[0].reshape(NW, NCHUNK, CH)
    col = ei[1].reshape(NW, NCHUNK, CH)
    x_full = jnp.concatenate([x, emb], axis=0)

    ones_ch = jnp.ones((CH, 16), jnp.float32)
    z1 = jnp.zeros((RPT, 16), jnp.float32)
    zfs = jnp.zeros((RPT, FS), jnp.float32)

    deg_parts = _deg_kernel(col, ones_ch, z1).reshape(NC, N, 16)[:, :, 0]
    dis = _dis_tc(deg_parts).reshape(N, 1)

    def layer(xin, W, b, relu):
        outs = []
        for k in range(W.shape[1] // FS):
            g = _mm_tc(dis, xin, W[:, k * FS:(k + 1) * FS])
            S = _prop(g, row, col, zfs).reshape(NC, N, FS)
            outs.append(_comb_tc(S, g, dis,
                                 b[k * FS:(k + 1) * FS].reshape(1, -1), relu))
        return jnp.concatenate(outs, axis=1) if len(outs) > 1 else outs[0]

    x1 = layer(x_full, W1, b1, True)
    x2 = layer(x1, W2, b2, True)
    x3 = layer(x2, W3, b3, False)

    return jnp.concatenate([x3, x2, x1], axis=-1)


# speedup vs baseline: 16.4247x; 1.1524x over previous
import functools

import jax
import jax.numpy as jnp
from jax import lax
from jax.experimental import pallas as pl
from jax.experimental.pallas import tpu as pltpu
from jax.experimental.pallas import tpu_sc as plsc

N = 10000
E = 320000
NC, NS = 2, 16
NW = NC * NS
EW = E // NW
CH = 100
NCHUNK = EW // CH
FS = 64
RPT = N // NS

_mesh = plsc.VectorSubcoreMesh(
    core_axis_name="c", subcore_axis_name="s", num_cores=NC, num_subcores=NS)


@functools.partial(
    pl.kernel,
    out_type=jax.ShapeDtypeStruct((NW, RPT, 16), jnp.float32),
    mesh=_mesh,
    compiler_params=pltpu.CompilerParams(use_tc_tiling_on_sc=False),
    scratch_types=[
        pltpu.VMEM((NCHUNK, CH), jnp.int32),
        pltpu.VMEM((CH, 16), jnp.float32),
        pltpu.VMEM_SHARED((N, 16), jnp.float32),
    ],
)
def _deg_kernel(col_hbm, ones_hbm, zeros_hbm, out_hbm, cidx, ones_v, acc):
    c = lax.axis_index("c")
    s = lax.axis_index("s")
    w = c * NS + s
    pltpu.sync_copy(zeros_hbm, acc.at[pl.ds(s * RPT, RPT)])
    pltpu.sync_copy(col_hbm.at[w], cidx)
    pltpu.sync_copy(ones_hbm, ones_v)
    plsc.subcore_barrier()

    def body(j, carry):
        pltpu.sync_copy(ones_v, acc.at[cidx.at[j]], add=True)
        return carry

    lax.fori_loop(0, NCHUNK, body, 0)
    plsc.subcore_barrier()
    pltpu.sync_copy(acc.at[pl.ds(s * RPT, RPT)], out_hbm.at[w])


def _make_propagate(F):
    NB = NCHUNK // 4

    @functools.partial(
        pl.kernel,
        out_type=jax.ShapeDtypeStruct((NW, RPT, F), jnp.float32),
        mesh=_mesh,
        compiler_params=pltpu.CompilerParams(use_tc_tiling_on_sc=False),
        scratch_types=[
            pltpu.VMEM((NCHUNK, CH), jnp.int32),
            pltpu.VMEM((NCHUNK, CH), jnp.int32),
            pltpu.VMEM((CH, F), jnp.float32),
            pltpu.VMEM((CH, F), jnp.float32),
            pltpu.VMEM((CH, F), jnp.float32),
            pltpu.VMEM((CH, F), jnp.float32),
            pltpu.VMEM_SHARED((N, F), jnp.float32),
        ] + [pltpu.SemaphoreType.DMA] * 8,
    )
    def _prop(g_hbm, row_hbm, col_hbm, out_hbm,
              ridx, cidx, b0, b1, b2, b3, acc,
              g0, g1, g2, g3, s0, s1, s2, s3):
        bufs = [b0, b1, b2, b3]
        gs = [g0, g1, g2, g3]
        ss = [s0, s1, s2, s3]
        c = lax.axis_index("c")
        s = lax.axis_index("s")
        w = c * NS + s
        base_r = s * RPT

        def zbody(i, carry):
            for k in range(F // 16):
                b0[i, pl.ds(k * 16, 16)] = jnp.zeros((16,), jnp.float32)
            return carry
        lax.fori_loop(0, CH, zbody, 0)
        for t in range(RPT // CH):
            pltpu.sync_copy(b0, acc.at[pl.ds(base_r + t * CH, CH)])
        rem = RPT % CH
        if rem:
            pltpu.sync_copy(b0.at[pl.ds(0, rem)],
                            acc.at[pl.ds(base_r + (RPT // CH) * CH, rem)])

        pltpu.sync_copy(row_hbm.at[w], ridx)
        pltpu.sync_copy(col_hbm.at[w], cidx)
        for k in range(4):
            pltpu.async_copy(g_hbm.at[ridx.at[k]], bufs[k], gs[k])
        plsc.subcore_barrier()

        def body(i, carry):
            j0 = 4 * i
            descs = []
            for k in range(4):
                pltpu.make_async_copy(
                    g_hbm.at[ridx.at[0]], bufs[k], gs[k]).wait()
                descs.append(pltpu.async_copy(
                    bufs[k], acc.at[cidx.at[j0 + k]], ss[k], add=True))
            for k in range(4):
                descs[k].wait()
                pltpu.async_copy(g_hbm.at[ridx.at[j0 + 4 + k]], bufs[k], gs[k])
            return carry

        lax.fori_loop(0, NB - 1, body, 0)
        j0 = 4 * (NB - 1)
        descs = []
        for k in range(4):
            pltpu.make_async_copy(g_hbm.at[ridx.at[0]], bufs[k], gs[k]).wait()
            descs.append(pltpu.async_copy(
                bufs[k], acc.at[cidx.at[j0 + k]], ss[k], add=True))
        for k in range(4):
            descs[k].wait()
        plsc.subcore_barrier()
        pltpu.sync_copy(acc.at[pl.ds(base_r, RPT)], out_hbm.at[w])

    return _prop


_prop = _make_propagate(FS)


def _dis_body(p_ref, o_ref):
    o_ref[...] = lax.rsqrt(1.0 + p_ref[0:1] + p_ref[1:2])


def _dis_tc(partials):
    return pl.pallas_call(
        _dis_body,
        out_shape=jax.ShapeDtypeStruct((1, N), jnp.float32),
    )(partials)


def _mm_body(dis_ref, x_ref, w_ref, o_ref):
    o_ref[...] = jnp.dot(dis_ref[...] * x_ref[...], w_ref[...],
                         preferred_element_type=jnp.float32)


def _mm_tc(dis, x, W, rb=2000):
    fin, fout = W.shape
    return pl.pallas_call(
        _mm_body,
        grid=(N // rb,),
        in_specs=[
            pl.BlockSpec((rb, 1), lambda i: (i, 0)),
            pl.BlockSpec((rb, fin), lambda i: (i, 0)),
            pl.BlockSpec((fin, fout), lambda i: (0, 0)),
        ],
        out_specs=pl.BlockSpec((rb, fout), lambda i: (i, 0)),
        out_shape=jax.ShapeDtypeStruct((N, fout), jnp.float32),
    )(dis, x, W)


def _comb_body(relu, s_ref, g_ref, dis_ref, b_ref, o_ref):
    v = dis_ref[...] * (s_ref[0] + s_ref[1] + g_ref[...]) + b_ref[...]
    o_ref[...] = jnp.maximum(v, 0.0) if relu else v


def _comb_tc(S, g, dis, b, relu, rb=2000):
    fout = g.shape[1]
    return pl.pallas_call(
        functools.partial(_comb_body, relu),
        grid=(N // rb,),
        in_specs=[
            pl.BlockSpec((NC, rb, fout), lambda i: (0, i, 0)),
            pl.BlockSpec((rb, fout), lambda i: (i, 0)),
            pl.BlockSpec((rb, 1), lambda i: (i, 0)),
            pl.BlockSpec((1, fout), lambda i: (0, 0)),
        ],
        out_specs=pl.BlockSpec((rb, fout), lambda i: (i, 0)),
        out_shape=jax.ShapeDtypeStruct((N, fout), jnp.float32),
    )(S, g, dis, b)


def kernel(x, edge_index, emb, W1, b1, W2, b2, W3, b3):
    ei = edge_index.astype(jnp.int32)
    row = ei[0].reshape(NW, NCHUNK, CH)
    col = ei[1].reshape(NW, NCHUNK, CH)
    x_full = jnp.concatenate([x, emb], axis=0)

    ones_ch = jnp.ones((CH, 16), jnp.float32)
    z1 = jnp.zeros((RPT, 16), jnp.float32)

    deg_parts = _deg_kernel(col, ones_ch, z1).reshape(NC, N, 16)[:, :, 0]
    dis = _dis_tc(deg_parts).reshape(N, 1)

    def layer(xin, W, b, relu):
        outs = []
        for k in range(W.shape[1] // FS):
            g = _mm_tc(dis, xin, W[:, k * FS:(k + 1) * FS])
            S = _prop(g, row, col).reshape(NC, N, FS)
            outs.append(_comb_tc(S, g, dis,
                                 b[k * FS:(k + 1) * FS].reshape(1, -1), relu))
        return jnp.concatenate(outs, axis=1) if len(outs) > 1 else outs[0]

    x1 = layer(x_full, W1, b1, True)
    x2 = layer(x1, W2, b2, True)
    x3 = layer(x2, W3, b3, False)

    return jnp.concatenate([x3, x2, x1], axis=-1)


# speedup vs baseline: 18.5360x; 1.1285x over previous
import functools

import jax
import jax.numpy as jnp
from jax import lax
from jax.experimental import pallas as pl
from jax.experimental.pallas import tpu as pltpu
from jax.experimental.pallas import tpu_sc as plsc

N = 10000
E = 320000
NC, NS = 2, 16
NW = NC * NS
EW = E // NW
CH = 100
NCHUNK = EW // CH
FS = 64
RPT = N // NS

_mesh = plsc.VectorSubcoreMesh(
    core_axis_name="c", subcore_axis_name="s", num_cores=NC, num_subcores=NS)


@functools.partial(
    pl.kernel,
    out_type=jax.ShapeDtypeStruct((NW, RPT, 16), jnp.float32),
    mesh=_mesh,
    compiler_params=pltpu.CompilerParams(use_tc_tiling_on_sc=False),
    scratch_types=[
        pltpu.VMEM((NCHUNK, CH), jnp.int32),
        pltpu.VMEM((CH, 16), jnp.float32),
        pltpu.VMEM_SHARED((N, 16), jnp.float32),
    ],
)
def _deg_kernel(col_hbm, ones_hbm, zeros_hbm, out_hbm, cidx, ones_v, acc):
    c = lax.axis_index("c")
    s = lax.axis_index("s")
    w = c * NS + s
    pltpu.sync_copy(zeros_hbm, acc.at[pl.ds(s * RPT, RPT)])
    pltpu.sync_copy(col_hbm.at[w], cidx)
    pltpu.sync_copy(ones_hbm, ones_v)
    plsc.subcore_barrier()

    def body(j, carry):
        pltpu.sync_copy(ones_v, acc.at[cidx.at[j]], add=True)
        return carry

    lax.fori_loop(0, NCHUNK, body, 0)
    plsc.subcore_barrier()
    pltpu.sync_copy(acc.at[pl.ds(s * RPT, RPT)], out_hbm.at[w])


def _make_propagate(F):
    NB = NCHUNK // 4

    @functools.partial(
        pl.kernel,
        out_type=jax.ShapeDtypeStruct((NW, RPT, F), jnp.float32),
        mesh=_mesh,
        compiler_params=pltpu.CompilerParams(use_tc_tiling_on_sc=False),
        scratch_types=[
            pltpu.VMEM((NCHUNK, CH), jnp.int32),
            pltpu.VMEM((NCHUNK, CH), jnp.int32),
            pltpu.VMEM((CH, F), jnp.float32),
            pltpu.VMEM((CH, F), jnp.float32),
            pltpu.VMEM((CH, F), jnp.float32),
            pltpu.VMEM((CH, F), jnp.float32),
            pltpu.VMEM_SHARED((N, F), jnp.float32),
        ] + [pltpu.SemaphoreType.DMA] * 8,
    )
    def _prop(g_hbm, row_hbm, col_hbm, out_hbm,
              ridx, cidx, b0, b1, b2, b3, acc,
              g0, g1, g2, g3, s0, s1, s2, s3):
        bufs = [b0, b1, b2, b3]
        gs = [g0, g1, g2, g3]
        ss = [s0, s1, s2, s3]
        c = lax.axis_index("c")
        s = lax.axis_index("s")
        w = c * NS + s
        base_r = s * RPT

        def zbody(i, carry):
            for k in range(F // 16):
                b0[i, pl.ds(k * 16, 16)] = jnp.zeros((16,), jnp.float32)
            return carry
        lax.fori_loop(0, CH, zbody, 0)
        for t in range(RPT // CH):
            pltpu.sync_copy(b0, acc.at[pl.ds(base_r + t * CH, CH)])
        rem = RPT % CH
        if rem:
            pltpu.sync_copy(b0.at[pl.ds(0, rem)],
                            acc.at[pl.ds(base_r + (RPT // CH) * CH, rem)])

        pltpu.sync_copy(row_hbm.at[w], ridx)
        pltpu.sync_copy(col_hbm.at[w], cidx)
        for k in range(4):
            pltpu.async_copy(g_hbm.at[ridx.at[k]], bufs[k], gs[k])
        plsc.subcore_barrier()

        def body(i, carry):
            j0 = 4 * i
            descs = []
            for k in range(4):
                pltpu.make_async_copy(
                    g_hbm.at[ridx.at[0]], bufs[k], gs[k]).wait()
                descs.append(pltpu.async_copy(
                    bufs[k], acc.at[cidx.at[j0 + k]], ss[k], add=True))
            for k in range(4):
                descs[k].wait()
                pltpu.async_copy(g_hbm.at[ridx.at[j0 + 4 + k]], bufs[k], gs[k])
            return carry

        lax.fori_loop(0, NB - 1, body, 0)
        j0 = 4 * (NB - 1)
        descs = []
        for k in range(4):
            pltpu.make_async_copy(g_hbm.at[ridx.at[0]], bufs[k], gs[k]).wait()
            descs.append(pltpu.async_copy(
                bufs[k], acc.at[cidx.at[j0 + k]], ss[k], add=True))
        for k in range(4):
            descs[k].wait()
        plsc.subcore_barrier()
        pltpu.sync_copy(acc.at[pl.ds(base_r, RPT)], out_hbm.at[w])

    return _prop


_prop = _make_propagate(FS)


def _dis_body(p_ref, o_ref):
    o_ref[...] = lax.rsqrt(1.0 + p_ref[0:1] + p_ref[1:2])


def _dis_tc(partials):
    return pl.pallas_call(
        _dis_body,
        out_shape=jax.ShapeDtypeStruct((1, N), jnp.float32),
    )(partials)


def _mm_body(widths, col0, dis_ref, *refs):
    xs = refs[:len(widths)]
    w_ref = refs[len(widths)]
    o_ref = refs[len(widths) + 1]
    acc = None
    off = 0
    for x_ref, wd in zip(xs, widths):
        part = jnp.dot(dis_ref[...] * x_ref[...],
                       w_ref[off:off + wd, col0:col0 + FS],
                       preferred_element_type=jnp.float32)
        acc = part if acc is None else acc + part
        off += wd
    o_ref[...] = acc


def _mm_tc(dis, xs, W, col0, rb=2000):
    widths = tuple(xx.shape[1] for xx in xs)
    return pl.pallas_call(
        functools.partial(_mm_body, widths, col0),
        grid=(N // rb,),
        in_specs=[pl.BlockSpec((rb, 1), lambda i: (i, 0))]
        + [pl.BlockSpec((rb, wd), lambda i: (i, 0)) for wd in widths]
        + [pl.BlockSpec(W.shape, lambda i: (0, 0))],
        out_specs=pl.BlockSpec((rb, FS), lambda i: (i, 0)),
        out_shape=jax.ShapeDtypeStruct((N, FS), jnp.float32),
    )(dis, *xs, W)


def _comb_body(relu, s_ref, g_ref, dis_ref, b_ref, o_ref):
    v = dis_ref[...] * (s_ref[0] + s_ref[1] + g_ref[...]) + b_ref[...]
    if relu:
        v = jnp.maximum(v, 0.0)
    o_ref[...] = v


def _comb_tc(S, g, dis, b, relu, rb=2000):
    return pl.pallas_call(
        functools.partial(_comb_body, relu),
        grid=(N // rb,),
        in_specs=[
            pl.BlockSpec((NC, rb, FS), lambda i: (0, i, 0)),
            pl.BlockSpec((rb, FS), lambda i: (i, 0)),
            pl.BlockSpec((rb, 1), lambda i: (i, 0)),
            pl.BlockSpec((1, FS), lambda i: (0, 0)),
        ],
        out_specs=pl.BlockSpec((rb, FS), lambda i: (i, 0)),
        out_shape=jax.ShapeDtypeStruct((N, FS), jnp.float32),
    )(S, g, dis, b)


def _pack_body(*refs):
    o_ref = refs[-1]
    o_ref[...] = jnp.concatenate([r[...] for r in refs[:-1]], axis=1)


def _pack_tc(slices, rb=2000):
    return pl.pallas_call(
        _pack_body,
        grid=(N // rb,),
        in_specs=[pl.BlockSpec((rb, FS), lambda i: (i, 0)) for _ in slices],
        out_specs=pl.BlockSpec((rb, 7 * FS), lambda i: (i, 0)),
        out_shape=jax.ShapeDtypeStruct((N, 7 * FS), jnp.float32),
    )(*slices)


def kernel(x, edge_index, emb, W1, b1, W2, b2, W3, b3):
    ei = edge_index.astype(jnp.int32)
    row = ei[0].reshape(NW, NCHUNK, CH)
    col = ei[1].reshape(NW, NCHUNK, CH)
    x_full = jnp.concatenate([x, emb], axis=0)

    ones_ch = jnp.ones((CH, 16), jnp.float32)
    z1 = jnp.zeros((RPT, 16), jnp.float32)

    deg_parts = _deg_kernel(col, ones_ch, z1).reshape(NC, N, 16)[:, :, 0]
    dis = _dis_tc(deg_parts).reshape(N, 1)

    def layer(xs_in, W, b, relu):
        outs = []
        for k in range(W.shape[1] // FS):
            g = _mm_tc(dis, xs_in, W, k * FS)
            S = _prop(g, row, col).reshape(NC, N, FS)
            outs.append(_comb_tc(S, g, dis,
                                 b[k * FS:(k + 1) * FS].reshape(1, -1), relu))
        return outs

    x1s = layer([x_full], W1, b1, True)
    x2s = layer(x1s, W2, b2, True)
    x3s = layer(x2s, W3, b3, False)

    return _pack_tc(x3s + x2s + x1s)


# speedup vs baseline: 18.7088x; 1.0093x over previous
import functools

import jax
import jax.numpy as jnp
from jax import lax
from jax.experimental import pallas as pl
from jax.experimental.pallas import tpu as pltpu
from jax.experimental.pallas import tpu_sc as plsc

N = 10000
E = 320000
NC, NS = 2, 16
NW = NC * NS
EW = E // NW
CH = 125
NCHUNK = EW // CH
FS = 64
RPT = N // NS

_mesh = plsc.VectorSubcoreMesh(
    core_axis_name="c", subcore_axis_name="s", num_cores=NC, num_subcores=NS)


@functools.partial(
    pl.kernel,
    out_type=jax.ShapeDtypeStruct((NW, RPT, 16), jnp.float32),
    mesh=_mesh,
    compiler_params=pltpu.CompilerParams(use_tc_tiling_on_sc=False),
    scratch_types=[
        pltpu.VMEM((NCHUNK, CH), jnp.int32),
        pltpu.VMEM((CH, 16), jnp.float32),
        pltpu.VMEM_SHARED((N, 16), jnp.float32),
    ],
)
def _deg_kernel(col_hbm, ones_hbm, zeros_hbm, out_hbm, cidx, ones_v, acc):
    c = lax.axis_index("c")
    s = lax.axis_index("s")
    w = c * NS + s
    pltpu.sync_copy(zeros_hbm, acc.at[pl.ds(s * RPT, RPT)])
    pltpu.sync_copy(col_hbm.at[w], cidx)
    pltpu.sync_copy(ones_hbm, ones_v)
    plsc.subcore_barrier()

    def body(j, carry):
        pltpu.sync_copy(ones_v, acc.at[cidx.at[j]], add=True)
        return carry

    lax.fori_loop(0, NCHUNK, body, 0)
    plsc.subcore_barrier()
    pltpu.sync_copy(acc.at[pl.ds(s * RPT, RPT)], out_hbm.at[w])


def _make_propagate(F):
    NB = NCHUNK // 4

    @functools.partial(
        pl.kernel,
        out_type=jax.ShapeDtypeStruct((NW, RPT, F), jnp.float32),
        mesh=_mesh,
        compiler_params=pltpu.CompilerParams(use_tc_tiling_on_sc=False),
        scratch_types=[
            pltpu.VMEM((NCHUNK, CH), jnp.int32),
            pltpu.VMEM((NCHUNK, CH), jnp.int32),
            pltpu.VMEM((CH, F), jnp.float32),
            pltpu.VMEM((CH, F), jnp.float32),
            pltpu.VMEM((CH, F), jnp.float32),
            pltpu.VMEM((CH, F), jnp.float32),
            pltpu.VMEM_SHARED((N, F), jnp.float32),
        ] + [pltpu.SemaphoreType.DMA] * 8,
    )
    def _prop(g_hbm, row_hbm, col_hbm, out_hbm,
              ridx, cidx, b0, b1, b2, b3, acc,
              g0, g1, g2, g3, s0, s1, s2, s3):
        bufs = [b0, b1, b2, b3]
        gs = [g0, g1, g2, g3]
        ss = [s0, s1, s2, s3]
        c = lax.axis_index("c")
        s = lax.axis_index("s")
        w = c * NS + s
        base_r = s * RPT

        def zbody(i, carry):
            for k in range(F // 16):
                b0[i, pl.ds(k * 16, 16)] = jnp.zeros((16,), jnp.float32)
            return carry
        lax.fori_loop(0, CH, zbody, 0)
        for t in range(RPT // CH):
            pltpu.sync_copy(b0, acc.at[pl.ds(base_r + t * CH, CH)])
        rem = RPT % CH
        if rem:
            pltpu.sync_copy(b0.at[pl.ds(0, rem)],
                            acc.at[pl.ds(base_r + (RPT // CH) * CH, rem)])

        pltpu.sync_copy(row_hbm.at[w], ridx)
        pltpu.sync_copy(col_hbm.at[w], cidx)
        for k in range(4):
            pltpu.async_copy(g_hbm.at[ridx.at[k]], bufs[k], gs[k])
        plsc.subcore_barrier()

        def body(i, carry):
            j0 = 4 * i
            descs = []
            for k in range(4):
                pltpu.make_async_copy(
                    g_hbm.at[ridx.at[0]], bufs[k], gs[k]).wait()
                descs.append(pltpu.async_copy(
                    bufs[k], acc.at[cidx.at[j0 + k]], ss[k], add=True))
            for k in range(4):
                descs[k].wait()
                pltpu.async_copy(g_hbm.at[ridx.at[j0 + 4 + k]], bufs[k], gs[k])
            return carry

        lax.fori_loop(0, NB - 1, body, 0)
        j0 = 4 * (NB - 1)
        descs = []
        for k in range(4):
            pltpu.make_async_copy(g_hbm.at[ridx.at[0]], bufs[k], gs[k]).wait()
            descs.append(pltpu.async_copy(
                bufs[k], acc.at[cidx.at[j0 + k]], ss[k], add=True))
        for k in range(4):
            descs[k].wait()
        plsc.subcore_barrier()
        pltpu.sync_copy(acc.at[pl.ds(base_r, RPT)], out_hbm.at[w])

    return _prop


_prop = _make_propagate(FS)


def _dis_body(p_ref, o_ref):
    o_ref[...] = lax.rsqrt(1.0 + p_ref[0:1] + p_ref[1:2])


def _dis_tc(partials):
    return pl.pallas_call(
        _dis_body,
        out_shape=jax.ShapeDtypeStruct((1, N), jnp.float32),
    )(partials)


def _mm_body(widths, col0, dis_ref, *refs):
    xs = refs[:len(widths)]
    w_ref = refs[len(widths)]
    o_ref = refs[len(widths) + 1]
    acc = None
    off = 0
    for x_ref, wd in zip(xs, widths):
        part = jnp.dot(dis_ref[...] * x_ref[...],
                       w_ref[off:off + wd, col0:col0 + FS],
                       preferred_element_type=jnp.float32)
        acc = part if acc is None else acc + part
        off += wd
    o_ref[...] = acc


def _mm_tc(dis, xs, W, col0, rb=2000):
    widths = tuple(xx.shape[1] for xx in xs)
    return pl.pallas_call(
        functools.partial(_mm_body, widths, col0),
        grid=(N // rb,),
        in_specs=[pl.BlockSpec((rb, 1), lambda i: (i, 0))]
        + [pl.BlockSpec((rb, wd), lambda i: (i, 0)) for wd in widths]
        + [pl.BlockSpec(W.shape, lambda i: (0, 0))],
        out_specs=pl.BlockSpec((rb, FS), lambda i: (i, 0)),
        out_shape=jax.ShapeDtypeStruct((N, FS), jnp.float32),
    )(dis, *xs, W)


def _comb_body(relu, s_ref, g_ref, dis_ref, b_ref, o_ref):
    v = dis_ref[...] * (s_ref[0] + s_ref[1] + g_ref[...]) + b_ref[...]
    if relu:
        v = jnp.maximum(v, 0.0)
    o_ref[...] = v


def _comb_tc(S, g, dis, b, relu, rb=2000):
    return pl.pallas_call(
        functools.partial(_comb_body, relu),
        grid=(N // rb,),
        in_specs=[
            pl.BlockSpec((NC, rb, FS), lambda i: (0, i, 0)),
            pl.BlockSpec((rb, FS), lambda i: (i, 0)),
            pl.BlockSpec((rb, 1), lambda i: (i, 0)),
            pl.BlockSpec((1, FS), lambda i: (0, 0)),
        ],
        out_specs=pl.BlockSpec((rb, FS), lambda i: (i, 0)),
        out_shape=jax.ShapeDtypeStruct((N, FS), jnp.float32),
    )(S, g, dis, b)


def _pack_body(*refs):
    o_ref = refs[-1]
    o_ref[...] = jnp.concatenate([r[...] for r in refs[:-1]], axis=1)


def _pack_tc(slices, rb=2000):
    return pl.pallas_call(
        _pack_body,
        grid=(N // rb,),
        in_specs=[pl.BlockSpec((rb, FS), lambda i: (i, 0)) for _ in slices],
        out_specs=pl.BlockSpec((rb, 7 * FS), lambda i: (i, 0)),
        out_shape=jax.ShapeDtypeStruct((N, 7 * FS), jnp.float32),
    )(*slices)


def kernel(x, edge_index, emb, W1, b1, W2, b2, W3, b3):
    ei = edge_index.astype(jnp.int32)
    row = ei[0].reshape(NW, NCHUNK, CH)
    col = ei[1].reshape(NW, NCHUNK, CH)
    x_full = jnp.concatenate([x, emb], axis=0)

    ones_ch = jnp.ones((CH, 16), jnp.float32)
    z1 = jnp.zeros((RPT, 16), jnp.float32)

    deg_parts = _deg_kernel(col, ones_ch, z1).reshape(NC, N, 16)[:, :, 0]
    dis = _dis_tc(deg_parts).reshape(N, 1)

    def layer(xs_in, W, b, relu):
        outs = []
        for k in range(W.shape[1] // FS):
            g = _mm_tc(dis, xs_in, W, k * FS)
            S = _prop(g, row, col).reshape(NC, N, FS)
            outs.append(_comb_tc(S, g, dis,
                                 b[k * FS:(k + 1) * FS].reshape(1, -1), relu))
        return outs

    x1s = layer([x_full], W1, b1, True)
    x2s = layer(x1s, W2, b2, True)
    x3s = layer(x2s, W3, b3, False)

    return _pack_tc(x3s + x2s + x1s)
